# Initial kernel scaffold; baseline (speedup 1.0000x reference)
import jax
import jax.numpy as jnp
from jax.experimental import pallas as pl


def kernel(fake, history, swap_mask, swap_idx):
    raise NotImplementedError("write your pallas kernel here")


# speedup vs baseline: 31.8531x; 31.8531x over previous
import jax
import jax.numpy as jnp
from jax import lax
from jax.experimental import pallas as pl
from jax.experimental.pallas import tpu as pltpu
from jax.experimental.pallas import tpu_sc as plsc

_HIST = 4096
_D = 16384
_B = 64
_NC = 2
_NS = 16
_NW = _NC * _NS
_RPW = _B // _NW
_L = 16


def _sc_body(fake_hbm, hist_hbm, mask_hbm, idx_hbm, out_hbm,
             idx_v, mask_v, buf_v):
    wid = lax.axis_index("s") * _NC + lax.axis_index("c")
    pltpu.sync_copy(idx_hbm, idx_v)
    pltpu.sync_copy(mask_hbm, mask_v)

    jv0 = lax.iota(jnp.int32, _L)
    neg1 = jnp.full((_L,), -1, jnp.int32)

    for r in range(_RPW):
        i = wid * _RPW + r
        idx_acc = neg1
        mask_acc = neg1
        for k in range(_B // _L):
            jv = jv0 + (k * _L)
            onehot = jv == i
            idx_blk = idx_v[pl.ds(k * _L, _L)]
            mask_blk = mask_v[pl.ds(k * _L, _L)]
            idx_acc = jnp.maximum(idx_acc, jnp.where(onehot, idx_blk, neg1))
            mask_acc = jnp.maximum(mask_acc, jnp.where(onehot, mask_blk, neg1))
        idx_i = jnp.max(idx_acc)
        mask_i = jnp.max(mask_acc)

        best_acc = neg1
        for k in range(_B // _L):
            jv = jv0 + (k * _L)
            idx_blk = idx_v[pl.ds(k * _L, _L)]
            mask_blk = mask_v[pl.ds(k * _L, _L)]
            hit = (idx_blk == idx_i) & (mask_blk == 1) & (jv < i)
            best_acc = jnp.maximum(best_acc, jnp.where(hit, jv, neg1))
        best = jnp.max(best_acc)

        use_hist = (mask_i == 1) & (best < 0)
        frow = jnp.where(mask_i == 1, jnp.maximum(best, 0), i)

        @pl.when(use_hist)
        def _():
            pltpu.sync_copy(hist_hbm.at[idx_i], buf_v.at[r])

        @pl.when(jnp.logical_not(use_hist))
        def _():
            pltpu.sync_copy(fake_hbm.at[frow], buf_v.at[r])

        pltpu.sync_copy(buf_v.at[r], out_hbm.at[i])


def kernel(fake, history, swap_mask, swap_idx):
    mesh = plsc.VectorSubcoreMesh(core_axis_name="c", subcore_axis_name="s")
    f = pl.kernel(
        _sc_body,
        out_type=jax.ShapeDtypeStruct((_B, _D), jnp.float32),
        mesh=mesh,
        compiler_params=pltpu.CompilerParams(needs_layout_passes=False),
        scratch_types=[
            pltpu.VMEM((_B,), jnp.int32),
            pltpu.VMEM((_B,), jnp.int32),
            pltpu.VMEM((_RPW, _D), jnp.float32),
        ],
    )
    return f(fake, history, swap_mask, swap_idx)
